# baseline (speedup 1.0000x reference)
import functools
import numpy as np
import jax
import jax.numpy as jnp
from jax import lax
from jax.experimental import pallas as pl
from jax.experimental.pallas import tpu as pltpu
from jax.experimental.pallas import tpu_sc as plsc

L = 1024
F = 520
TOPK = 6
NEG = -3.0e38


def _dft_mats():
    l = np.arange(L, dtype=np.int64)
    f = np.arange(F, dtype=np.int64)
    m = (np.outer(f, l) % L).astype(np.float64) * (2.0 * np.pi / L)
    c = np.cos(m)
    s = np.sin(m)
    valid = (f <= L // 2).astype(np.float64)[:, None]
    wc = (c * valid).astype(np.float32)
    ws = (-s * valid).astype(np.float32)
    wf = np.where((f == 0) | (f == L // 2), 1.0, 2.0) / L
    ic = (c.T * wf[None, :] * valid.T).astype(np.float32)
    isn = (-s.T * wf[None, :] * valid.T).astype(np.float32)
    return wc, ws, ic, isn


_WC, _WS, _IC, _ISN = _dft_mats()


def _split(x):
    hi = x.astype(jnp.bfloat16)
    lo = (x - hi.astype(jnp.float32)).astype(jnp.bfloat16)
    return hi, lo


def _corr_body(q_ref, k_ref, wc_ref, ws_ref, ic_ref, isn_ref,
               corr_ref, w_ref, d_ref):
    C = q_ref.shape[-1]
    dot = functools.partial(
        jax.lax.dot_general,
        dimension_numbers=(((1,), (0,)), ((), ())),
        preferred_element_type=jnp.float32)

    def dot3(a, b):
        ah, al = _split(a)
        bh, bl = _split(b)
        return dot(ah, bh) + (dot(ah, bl) + dot(al, bh))

    q = q_ref[0]
    k = k_ref[0]
    wc = wc_ref[...]
    ws = ws_ref[...]

    qr = dot3(wc, q)
    qi = dot3(ws, q)
    kr = dot3(wc, k)
    ki = dot3(ws, k)
    pr = qr * kr + qi * ki
    pi = qi * kr - qr * ki
    corr = dot3(ic_ref[...], pr) + dot3(isn_ref[...], pi)
    corr_ref[0] = corr

    riota = jax.lax.broadcasted_iota(jnp.int32, (L, C), 0)
    c = corr
    tops, delays = [], []
    for _ in range(TOPK):
        m = jnp.max(c, axis=0, keepdims=True)
        idx = jnp.min(jnp.where(c == m, riota, L), axis=0, keepdims=True)
        c = jnp.where(riota == idx, NEG, c)
        tops.append(m)
        delays.append(idx)

    es = [jnp.exp(w - tops[0]) for w in tops]
    tot = es[0]
    for e in es[1:]:
        tot = tot + e
    inv = 1.0 / tot

    zero_f = jnp.zeros((2, C), jnp.float32)
    zero_i = jnp.zeros((2, C), jnp.int32)
    w_ref[0] = jnp.concatenate([e * inv for e in es] + [zero_f], axis=0)
    d_ref[0] = jnp.concatenate(delays + [zero_i], axis=0)


def _corr_topk(q, k):
    BN, Lq, C = q.shape
    CB = 512
    blk = lambda i, j: (i, 0, j)
    fix = lambda i, j: (0, 0)
    return pl.pallas_call(
        _corr_body,
        grid=(BN, C // CB),
        in_specs=[
            pl.BlockSpec((1, L, CB), blk),
            pl.BlockSpec((1, L, CB), blk),
            pl.BlockSpec((F, L), fix),
            pl.BlockSpec((F, L), fix),
            pl.BlockSpec((L, F), fix),
            pl.BlockSpec((L, F), fix),
        ],
        out_specs=[
            pl.BlockSpec((1, L, CB), blk),
            pl.BlockSpec((1, 8, CB), blk),
            pl.BlockSpec((1, 8, CB), blk),
        ],
        out_shape=[
            jax.ShapeDtypeStruct((BN, L, C), jnp.float32),
            jax.ShapeDtypeStruct((BN, 8, C), jnp.float32),
            jax.ShapeDtypeStruct((BN, 8, C), jnp.int32),
        ],
    )(q, k, jnp.asarray(_WC), jnp.asarray(_WS),
      jnp.asarray(_IC), jnp.asarray(_ISN))


def _delay_agg(v, w, d):
    BN, Lq, C = v.shape
    info = plsc.get_sparse_core_info()
    NC, NS, NL = info.num_cores, info.num_subcores, info.num_lanes
    NW = NC * NS
    n_chunks = C // NL
    n_tasks = BN * n_chunks
    tasks_per_w = n_tasks // NW
    mesh = plsc.VectorSubcoreMesh(core_axis_name="c", subcore_axis_name="s")

    @functools.partial(
        pl.kernel,
        mesh=mesh,
        compiler_params=pltpu.CompilerParams(
            use_tc_tiling_on_sc=False, needs_layout_passes=False),
        out_type=jax.ShapeDtypeStruct((BN, Lq, C), jnp.float32),
        scratch_types=[
            pltpu.VMEM((2 * Lq, NL), jnp.float32),
            pltpu.VMEM((8, NL), jnp.float32),
            pltpu.VMEM((8, NL), jnp.int32),
            pltpu.VMEM((Lq, NL), jnp.float32),
        ],
    )
    def agg(v_hbm, w_hbm, d_hbm, out_hbm, vbuf, wbuf, dbuf, obuf):
        wid = lax.axis_index("s") * NC + lax.axis_index("c")
        lanes = jax.lax.broadcasted_iota(jnp.int32, (NL,), 0)
        U = 8
        for t in range(tasks_per_w):
            task = wid * tasks_per_w + t
            bn = task // n_chunks
            ch0 = (task % n_chunks) * NL
            pltpu.sync_copy(v_hbm.at[bn, :, pl.ds(ch0, NL)],
                            vbuf.at[pl.ds(0, Lq)])
            pltpu.sync_copy(v_hbm.at[bn, :, pl.ds(ch0, NL)],
                            vbuf.at[pl.ds(Lq, Lq)])
            pltpu.sync_copy(w_hbm.at[bn, :, pl.ds(ch0, NL)], wbuf)
            pltpu.sync_copy(d_hbm.at[bn, :, pl.ds(ch0, NL)], dbuf)
            wv = [wbuf[i] for i in range(TOPK)]

            def group(g, idxs):
                base = g * U
                for u in range(U):
                    acc = None
                    for i in range(TOPK):
                        gt = plsc.load_gather(vbuf, [idxs[i], lanes])
                        acc = gt * wv[i] if acc is None else acc + gt * wv[i]
                    obuf[base + u] = acc
                    idxs = tuple(x + 1 for x in idxs)
                return idxs

            lax.fori_loop(0, Lq // U, group,
                          tuple(dbuf[i] for i in range(TOPK)))
            pltpu.sync_copy(obuf, out_hbm.at[bn, :, pl.ds(ch0, NL)])

    return agg(v, w, d)


def kernel(queries, keys, values, attn_mask):
    B, N, Lq, H, E = queries.shape
    C = H * E
    BN = B * N
    q = queries.reshape(BN, Lq, C)
    k = keys.reshape(BN, Lq, C)
    v = values.reshape(BN, Lq, C)

    corr, w, d = _corr_topk(q, k)
    vout = _delay_agg(v, w, d)

    V = vout.reshape(B, N, Lq, H, E)
    corr_t = corr.reshape(B, N, Lq, H, E)
    return (V, corr_t)


# speedup vs baseline: 1.1322x; 1.1322x over previous
import functools
import numpy as np
import jax
import jax.numpy as jnp
from jax import lax
from jax.experimental import pallas as pl
from jax.experimental.pallas import tpu as pltpu
from jax.experimental.pallas import tpu_sc as plsc

L = 1024
F = 520
TOPK = 6
NEG = -3.0e38


def _dft_mats():
    l = np.arange(L, dtype=np.int64)
    f = np.arange(F, dtype=np.int64)
    m = (np.outer(f, l) % L).astype(np.float64) * (2.0 * np.pi / L)
    c = np.cos(m)
    s = np.sin(m)
    valid = (f <= L // 2).astype(np.float64)[:, None]
    wc = (c * valid).astype(np.float32)
    ws = (-s * valid).astype(np.float32)
    wf = np.where((f == 0) | (f == L // 2), 1.0, 2.0) / L
    ic = (c.T * wf[None, :] * valid.T).astype(np.float32)
    isn = (-s.T * wf[None, :] * valid.T).astype(np.float32)
    return wc, ws, ic, isn


_WC, _WS, _IC, _ISN = _dft_mats()


def _split(x):
    hi = x.astype(jnp.bfloat16)
    lo = (x - hi.astype(jnp.float32)).astype(jnp.bfloat16)
    return hi, lo


def _corr_body(q_ref, k_ref, wc_ref, ws_ref, ic_ref, isn_ref,
               corr_ref, w_ref, d_ref):
    C = q_ref.shape[-1]
    dot = functools.partial(
        jax.lax.dot_general,
        dimension_numbers=(((1,), (0,)), ((), ())),
        preferred_element_type=jnp.float32)

    def dot3(a, b):
        ah, al = _split(a)
        bh, bl = _split(b)
        return dot(ah, bh) + (dot(ah, bl) + dot(al, bh))

    q = q_ref[0]
    k = k_ref[0]
    wc = wc_ref[...]
    ws = ws_ref[...]

    qr = dot3(wc, q)
    qi = dot3(ws, q)
    kr = dot3(wc, k)
    ki = dot3(ws, k)
    pr = qr * kr + qi * ki
    pi = qi * kr - qr * ki
    corr = dot3(ic_ref[...], pr) + dot3(isn_ref[...], pi)
    corr_ref[0] = corr

    riota = jax.lax.broadcasted_iota(jnp.int32, (L, C), 0)
    c = corr
    tops, delays = [], []
    for _ in range(TOPK):
        m = jnp.max(c, axis=0, keepdims=True)
        idx = jnp.min(jnp.where(c == m, riota, L), axis=0, keepdims=True)
        c = jnp.where(riota == idx, NEG, c)
        tops.append(m)
        delays.append(idx)

    es = [jnp.exp(w - tops[0]) for w in tops]
    tot = es[0]
    for e in es[1:]:
        tot = tot + e
    inv = 1.0 / tot

    zero_f = jnp.zeros((2, C), jnp.float32)
    zero_i = jnp.zeros((2, C), jnp.int32)
    w_ref[0] = jnp.concatenate([e * inv for e in es] + [zero_f], axis=0)
    d_ref[0] = jnp.concatenate(delays + [zero_i], axis=0)


def _corr_topk(q, k):
    BN, Lq, C = q.shape
    CB = 512
    blk = lambda i, j: (i, 0, j)
    fix = lambda i, j: (0, 0)
    return pl.pallas_call(
        _corr_body,
        grid=(BN, C // CB),
        in_specs=[
            pl.BlockSpec((1, L, CB), blk),
            pl.BlockSpec((1, L, CB), blk),
            pl.BlockSpec((F, L), fix),
            pl.BlockSpec((F, L), fix),
            pl.BlockSpec((L, F), fix),
            pl.BlockSpec((L, F), fix),
        ],
        out_specs=[
            pl.BlockSpec((1, L, CB), blk),
            pl.BlockSpec((1, 8, CB), blk),
            pl.BlockSpec((1, 8, CB), blk),
        ],
        out_shape=[
            jax.ShapeDtypeStruct((BN, L, C), jnp.float32),
            jax.ShapeDtypeStruct((BN, 8, C), jnp.float32),
            jax.ShapeDtypeStruct((BN, 8, C), jnp.int32),
        ],
    )(q, k, jnp.asarray(_WC), jnp.asarray(_WS),
      jnp.asarray(_IC), jnp.asarray(_ISN))


def _delay_agg(v, w, d):
    BN, Lq, C = v.shape
    info = plsc.get_sparse_core_info()
    NC, NS, NL = info.num_cores, info.num_subcores, info.num_lanes
    NW = NC * NS
    n_chunks = C // NL
    n_tasks = BN * n_chunks
    tasks_per_w = n_tasks // NW
    mesh = plsc.VectorSubcoreMesh(core_axis_name="c", subcore_axis_name="s")

    CW = tasks_per_w * NL

    @functools.partial(
        pl.kernel,
        mesh=mesh,
        compiler_params=pltpu.CompilerParams(
            use_tc_tiling_on_sc=False, needs_layout_passes=False),
        out_type=jax.ShapeDtypeStruct((BN, Lq, C), jnp.float32),
        scratch_types=[
            pltpu.VMEM((2, Lq, NL), jnp.float32),
            pltpu.VMEM((2, Lq, NL), jnp.float32),
            pltpu.VMEM((8, CW), jnp.float32),
            pltpu.VMEM((8, CW), jnp.int32),
            pltpu.SemaphoreType.DMA,
            pltpu.SemaphoreType.DMA,
            pltpu.SemaphoreType.DMA,
            pltpu.SemaphoreType.DMA,
        ],
    )
    def agg(v_hbm, w_hbm, d_hbm, out_hbm, vbuf, obuf, wbuf, dbuf,
            si0, si1, so0, so1):
        wid = lax.axis_index("s") * NC + lax.axis_index("c")
        bn = wid // (C // CW)
        chb = (wid % (C // CW)) * CW
        lanes = jax.lax.broadcasted_iota(jnp.int32, (NL,), 0)
        sin = [si0, si1]
        sout = [so0, so1]
        pltpu.sync_copy(w_hbm.at[bn, :, pl.ds(chb, CW)], wbuf)
        pltpu.sync_copy(d_hbm.at[bn, :, pl.ds(chb, CW)], dbuf)

        def vin(t, buf):
            return pltpu.make_async_copy(
                v_hbm.at[bn, :, pl.ds(chb + t * NL, NL)],
                vbuf.at[buf], sin[buf])

        def vout(t, buf):
            return pltpu.make_async_copy(
                obuf.at[buf],
                out_hbm.at[bn, :, pl.ds(chb + t * NL, NL)], sout[buf])

        vin(0, 0).start()
        U = 8
        for t in range(tasks_per_w):
            cur = t % 2
            if t + 1 < tasks_per_w:
                vin(t + 1, 1 - cur).start()
            vin(t, cur).wait()
            if t >= 2:
                vout(t - 2, cur).wait()
            wv = [wbuf[i, pl.ds(t * NL, NL)] for i in range(TOPK)]
            vb = vbuf.at[cur]
            ob = obuf.at[cur]

            def group(g, idxs):
                base = g * U
                for u in range(U):
                    acc = None
                    for i in range(TOPK):
                        ix = jnp.bitwise_and(idxs[i], L - 1)
                        gt = plsc.load_gather(vb, [ix, lanes])
                        acc = gt * wv[i] if acc is None else acc + gt * wv[i]
                    ob[base + u] = acc
                    idxs = tuple(x + 1 for x in idxs)
                return idxs

            lax.fori_loop(0, Lq // U, group,
                          tuple(dbuf[i, pl.ds(t * NL, NL)]
                                for i in range(TOPK)))
            vout(t, cur).start()
        vout(tasks_per_w - 2, 0 if tasks_per_w % 2 == 0 else 1).wait()
        vout(tasks_per_w - 1, 1 if tasks_per_w % 2 == 0 else 0).wait()

    return agg(v, w, d)


def kernel(queries, keys, values, attn_mask):
    B, N, Lq, H, E = queries.shape
    C = H * E
    BN = B * N
    q = queries.reshape(BN, Lq, C)
    k = keys.reshape(BN, Lq, C)
    v = values.reshape(BN, Lq, C)

    corr, w, d = _corr_topk(q, k)
    vout = _delay_agg(v, w, d)

    V = vout.reshape(B, N, Lq, H, E)
    corr_t = corr.reshape(B, N, Lq, H, E)
    return (V, corr_t)


# speedup vs baseline: 1.3168x; 1.1631x over previous
import functools
import numpy as np
import jax
import jax.numpy as jnp
from jax import lax
from jax.experimental import pallas as pl
from jax.experimental.pallas import tpu as pltpu
from jax.experimental.pallas import tpu_sc as plsc

L = 1024
F = 520
TOPK = 6
NEG = -3.0e38


H2 = L // 2
FE = 264
FO = 256


def _dft_mats():
    j = np.arange(H2, dtype=np.int64)
    g = np.arange(FE, dtype=np.int64)
    h = np.arange(FO, dtype=np.int64)
    tau = np.arange(H2, dtype=np.int64)
    ev = np.minimum(g, 256)
    me = (np.outer(ev, j) % H2).astype(np.float64) * (2.0 * np.pi / H2)
    mo = (np.outer(2 * h + 1, j) % L).astype(np.float64) * (2.0 * np.pi / L)
    vg = (g <= 256).astype(np.float64)[:, None]
    we = (np.cos(me) * vg).astype(np.float32)
    ve = (-np.sin(me) * vg).astype(np.float32)
    wo = np.cos(mo).astype(np.float32)
    vo = (-np.sin(mo)).astype(np.float32)
    wf_e = np.where((ev == 0) | (ev == 256), 1.0, 2.0) / L
    te = (np.outer(tau, ev) % H2).astype(np.float64) * (2.0 * np.pi / H2)
    to = (np.outer(tau, 2 * h + 1) % L).astype(np.float64) * (2.0 * np.pi / L)
    ice = (np.cos(te) * wf_e[None, :] * vg.T).astype(np.float32)
    ise = (-np.sin(te) * wf_e[None, :] * vg.T).astype(np.float32)
    ico = (np.cos(to) * (2.0 / L)).astype(np.float32)
    iso = (-np.sin(to) * (2.0 / L)).astype(np.float32)
    return we, ve, wo, vo, ice, ise, ico, iso


_MATS = _dft_mats()


def _split(x):
    hi = x.astype(jnp.bfloat16)
    lo = (x - hi.astype(jnp.float32)).astype(jnp.bfloat16)
    return hi, lo


def _corr_body(q_ref, k_ref, we_ref, ve_ref, wo_ref, vo_ref,
               ice_ref, ise_ref, ico_ref, iso_ref,
               corr_ref, w_ref, d_ref):
    C = q_ref.shape[-1]
    dot = functools.partial(
        jax.lax.dot_general,
        dimension_numbers=(((1,), (0,)), ((), ())),
        preferred_element_type=jnp.float32)

    def dot3(a, b):
        ah, al = _split(a)
        bh, bl = _split(b)
        return dot(ah, bh) + (dot(ah, bl) + dot(al, bh))

    q = q_ref[0]
    k = k_ref[0]

    qp = q[:H2] + q[H2:]
    qm = q[:H2] - q[H2:]
    kp = k[:H2] + k[H2:]
    km = k[:H2] - k[H2:]
    qre = dot3(we_ref[...], qp)
    qie = dot3(ve_ref[...], qp)
    qro = dot3(wo_ref[...], qm)
    qio = dot3(vo_ref[...], qm)
    kre = dot3(we_ref[...], kp)
    kie = dot3(ve_ref[...], kp)
    kro = dot3(wo_ref[...], km)
    kio = dot3(vo_ref[...], km)
    pre = qre * kre + qie * kie
    pie = qie * kre - qre * kie
    pro = qro * kro + qio * kio
    pio = qio * kro - qro * kio
    ue = dot3(ice_ref[...], pre) + dot3(ise_ref[...], pie)
    uo = dot3(ico_ref[...], pro) + dot3(iso_ref[...], pio)
    corr = jnp.concatenate([ue + uo, ue - uo], axis=0)
    corr_ref[0] = corr

    riota = jax.lax.broadcasted_iota(jnp.int32, (L, C), 0)
    c = corr
    tops, delays = [], []
    for _ in range(TOPK):
        m = jnp.max(c, axis=0, keepdims=True)
        idx = jnp.min(jnp.where(c == m, riota, L), axis=0, keepdims=True)
        c = jnp.where(riota == idx, NEG, c)
        tops.append(m)
        delays.append(idx)

    es = [jnp.exp(w - tops[0]) for w in tops]
    tot = es[0]
    for e in es[1:]:
        tot = tot + e
    inv = 1.0 / tot

    zero_f = jnp.zeros((2, C), jnp.float32)
    zero_i = jnp.zeros((2, C), jnp.int32)
    w_ref[0] = jnp.concatenate([e * inv for e in es] + [zero_f], axis=0)
    d_ref[0] = jnp.concatenate(delays + [zero_i], axis=0)


def _corr_topk(q, k):
    BN, Lq, C = q.shape
    CB = 512
    blk = lambda i, j: (i, 0, j)
    fix = lambda i, j: (0, 0)
    return pl.pallas_call(
        _corr_body,
        grid=(BN, C // CB),
        in_specs=[
            pl.BlockSpec((1, L, CB), blk),
            pl.BlockSpec((1, L, CB), blk),
        ] + [pl.BlockSpec(m.shape, fix) for m in _MATS],
        out_specs=[
            pl.BlockSpec((1, L, CB), blk),
            pl.BlockSpec((1, 8, CB), blk),
            pl.BlockSpec((1, 8, CB), blk),
        ],
        out_shape=[
            jax.ShapeDtypeStruct((BN, L, C), jnp.float32),
            jax.ShapeDtypeStruct((BN, 8, C), jnp.float32),
            jax.ShapeDtypeStruct((BN, 8, C), jnp.int32),
        ],
    )(q, k, *[jnp.asarray(m) for m in _MATS])


def _delay_agg(v, w, d):
    BN, Lq, C = v.shape
    info = plsc.get_sparse_core_info()
    NC, NS, NL = info.num_cores, info.num_subcores, info.num_lanes
    NW = NC * NS
    n_chunks = C // NL
    n_tasks = BN * n_chunks
    tasks_per_w = n_tasks // NW
    mesh = plsc.VectorSubcoreMesh(core_axis_name="c", subcore_axis_name="s")

    CW = tasks_per_w * NL

    @functools.partial(
        pl.kernel,
        mesh=mesh,
        compiler_params=pltpu.CompilerParams(
            use_tc_tiling_on_sc=False, needs_layout_passes=False),
        out_type=jax.ShapeDtypeStruct((BN, Lq, C), jnp.float32),
        scratch_types=[
            pltpu.VMEM((2, Lq, NL), jnp.float32),
            pltpu.VMEM((2, Lq, NL), jnp.float32),
            pltpu.VMEM((8, CW), jnp.float32),
            pltpu.VMEM((8, CW), jnp.int32),
            pltpu.SemaphoreType.DMA,
            pltpu.SemaphoreType.DMA,
            pltpu.SemaphoreType.DMA,
            pltpu.SemaphoreType.DMA,
        ],
    )
    def agg(v_hbm, w_hbm, d_hbm, out_hbm, vbuf, obuf, wbuf, dbuf,
            si0, si1, so0, so1):
        wid = lax.axis_index("s") * NC + lax.axis_index("c")
        bn = wid // (C // CW)
        chb = (wid % (C // CW)) * CW
        lanes = jax.lax.broadcasted_iota(jnp.int32, (NL,), 0)
        sin = [si0, si1]
        sout = [so0, so1]
        pltpu.sync_copy(w_hbm.at[bn, :, pl.ds(chb, CW)], wbuf)
        pltpu.sync_copy(d_hbm.at[bn, :, pl.ds(chb, CW)], dbuf)

        def vin(t, buf):
            return pltpu.make_async_copy(
                v_hbm.at[bn, :, pl.ds(chb + t * NL, NL)],
                vbuf.at[buf], sin[buf])

        def vout(t, buf):
            return pltpu.make_async_copy(
                obuf.at[buf],
                out_hbm.at[bn, :, pl.ds(chb + t * NL, NL)], sout[buf])

        vin(0, 0).start()
        U = 8
        for t in range(tasks_per_w):
            cur = t % 2
            if t + 1 < tasks_per_w:
                vin(t + 1, 1 - cur).start()
            vin(t, cur).wait()
            if t >= 2:
                vout(t - 2, cur).wait()
            wv = [wbuf[i, pl.ds(t * NL, NL)] for i in range(TOPK)]
            vb = vbuf.at[cur]
            ob = obuf.at[cur]

            def group(g, idxs):
                base = g * U
                for u in range(U):
                    acc = None
                    for i in range(TOPK):
                        ix = jnp.bitwise_and(idxs[i], L - 1)
                        gt = plsc.load_gather(vb, [ix, lanes])
                        acc = gt * wv[i] if acc is None else acc + gt * wv[i]
                    ob[base + u] = acc
                    idxs = tuple(x + 1 for x in idxs)
                return idxs

            lax.fori_loop(0, Lq // U, group,
                          tuple(dbuf[i, pl.ds(t * NL, NL)]
                                for i in range(TOPK)))
            vout(t, cur).start()
        vout(tasks_per_w - 2, 0 if tasks_per_w % 2 == 0 else 1).wait()
        vout(tasks_per_w - 1, 1 if tasks_per_w % 2 == 0 else 0).wait()

    return agg(v, w, d)


def kernel(queries, keys, values, attn_mask):
    B, N, Lq, H, E = queries.shape
    C = H * E
    BN = B * N
    q = queries.reshape(BN, Lq, C)
    k = keys.reshape(BN, Lq, C)
    v = values.reshape(BN, Lq, C)

    corr, w, d = _corr_topk(q, k)
    vout = _delay_agg(v, w, d)

    V = vout.reshape(B, N, Lq, H, E)
    corr_t = corr.reshape(B, N, Lq, H, E)
    return (V, corr_t)


# speedup vs baseline: 1.3380x; 1.0161x over previous
import functools
import numpy as np
import jax
import jax.numpy as jnp
from jax import lax
from jax.experimental import pallas as pl
from jax.experimental.pallas import tpu as pltpu
from jax.experimental.pallas import tpu_sc as plsc

L = 1024
F = 520
TOPK = 6
NEG = -3.0e38


H2 = L // 2
FE = 264
FO = 256


def _dft_mats():
    j = np.arange(H2, dtype=np.int64)
    g = np.arange(FE, dtype=np.int64)
    h = np.arange(FO, dtype=np.int64)
    tau = np.arange(H2, dtype=np.int64)
    ev = np.minimum(g, 256)
    me = (np.outer(ev, j) % H2).astype(np.float64) * (2.0 * np.pi / H2)
    mo = (np.outer(2 * h + 1, j) % L).astype(np.float64) * (2.0 * np.pi / L)
    vg = (g <= 256).astype(np.float64)[:, None]
    we = (np.cos(me) * vg).astype(np.float32)
    ve = (-np.sin(me) * vg).astype(np.float32)
    wo = np.cos(mo).astype(np.float32)
    vo = (-np.sin(mo)).astype(np.float32)
    wf_e = np.where((ev == 0) | (ev == 256), 1.0, 2.0) / L
    te = (np.outer(tau, ev) % H2).astype(np.float64) * (2.0 * np.pi / H2)
    to = (np.outer(tau, 2 * h + 1) % L).astype(np.float64) * (2.0 * np.pi / L)
    ice = (np.cos(te) * wf_e[None, :] * vg.T).astype(np.float32)
    ise = (-np.sin(te) * wf_e[None, :] * vg.T).astype(np.float32)
    ico = (np.cos(to) * (2.0 / L)).astype(np.float32)
    iso = (-np.sin(to) * (2.0 / L)).astype(np.float32)
    return we, ve, wo, vo, ice, ise, ico, iso


_MATS = _dft_mats()


def _split(x):
    hi = x.astype(jnp.bfloat16)
    lo = (x - hi.astype(jnp.float32)).astype(jnp.bfloat16)
    return hi, lo


def _corr_body(q_ref, k_ref, we_ref, ve_ref, wo_ref, vo_ref,
               ice_ref, ise_ref, ico_ref, iso_ref,
               corr_ref, w_ref, d_ref):
    C = q_ref.shape[-1]
    dot = functools.partial(
        jax.lax.dot_general,
        dimension_numbers=(((1,), (0,)), ((), ())),
        preferred_element_type=jnp.float32)

    def dot3(a, b):
        ah, al = _split(a)
        bh, bl = _split(b)
        return dot(ah, bh) + (dot(ah, bl) + dot(al, bh))

    SB = C // 2
    for sb in range(2):
        sl = pl.ds(sb * SB, SB)
        q = q_ref[0, :, sl]
        k = k_ref[0, :, sl]

        qp = q[:H2] + q[H2:]
        qm = q[:H2] - q[H2:]
        kp = k[:H2] + k[H2:]
        km = k[:H2] - k[H2:]
        qre = dot3(we_ref[...], qp)
        qie = dot3(ve_ref[...], qp)
        qro = dot3(wo_ref[...], qm)
        qio = dot3(vo_ref[...], qm)
        kre = dot3(we_ref[...], kp)
        kie = dot3(ve_ref[...], kp)
        kro = dot3(wo_ref[...], km)
        kio = dot3(vo_ref[...], km)
        pre = qre * kre + qie * kie
        pie = qie * kre - qre * kie
        pro = qro * kro + qio * kio
        pio = qio * kro - qro * kio
        ue = dot3(ice_ref[...], pre) + dot3(ise_ref[...], pie)
        uo = dot3(ico_ref[...], pro) + dot3(iso_ref[...], pio)
        corr = jnp.concatenate([ue + uo, ue - uo], axis=0)
        corr_ref[0, :, sl] = corr

        riota = jax.lax.broadcasted_iota(jnp.int32, (L, SB), 0)
        c = corr
        tops, delays = [], []
        for _ in range(TOPK):
            m = jnp.max(c, axis=0, keepdims=True)
            idx = jnp.min(jnp.where(c == m, riota, L), axis=0,
                          keepdims=True)
            c = jnp.where(riota == idx, NEG, c)
            tops.append(m)
            delays.append(idx)

        es = [jnp.exp(w - tops[0]) for w in tops]
        tot = es[0]
        for e in es[1:]:
            tot = tot + e
        inv = 1.0 / tot

        zero_f = jnp.zeros((2, SB), jnp.float32)
        zero_i = jnp.zeros((2, SB), jnp.int32)
        w_ref[0, :, sl] = jnp.concatenate(
            [e * inv for e in es] + [zero_f], axis=0)
        d_ref[0, :, sl] = jnp.concatenate(delays + [zero_i], axis=0)


def _corr_topk(q, k):
    BN, Lq, C = q.shape
    CB = 512
    blk = lambda i, j: (i, 0, j)
    fix = lambda i, j: (0, 0)
    return pl.pallas_call(
        _corr_body,
        grid=(BN, C // CB),
        in_specs=[
            pl.BlockSpec((1, L, CB), blk),
            pl.BlockSpec((1, L, CB), blk),
        ] + [pl.BlockSpec(m.shape, fix) for m in _MATS],
        out_specs=[
            pl.BlockSpec((1, L, CB), blk),
            pl.BlockSpec((1, 8, CB), blk),
            pl.BlockSpec((1, 8, CB), blk),
        ],
        out_shape=[
            jax.ShapeDtypeStruct((BN, L, C), jnp.float32),
            jax.ShapeDtypeStruct((BN, 8, C), jnp.float32),
            jax.ShapeDtypeStruct((BN, 8, C), jnp.int32),
        ],
    )(q, k, *[jnp.asarray(m) for m in _MATS])


def _delay_agg(v, w, d):
    BN, Lq, C = v.shape
    info = plsc.get_sparse_core_info()
    NC, NS, NL = info.num_cores, info.num_subcores, info.num_lanes
    NW = NC * NS
    n_chunks = C // NL
    n_tasks = BN * n_chunks
    tasks_per_w = n_tasks // NW
    mesh = plsc.VectorSubcoreMesh(core_axis_name="c", subcore_axis_name="s")

    CW = tasks_per_w * NL

    @functools.partial(
        pl.kernel,
        mesh=mesh,
        compiler_params=pltpu.CompilerParams(
            use_tc_tiling_on_sc=False, needs_layout_passes=False),
        out_type=jax.ShapeDtypeStruct((BN, Lq, C), jnp.float32),
        scratch_types=[
            pltpu.VMEM((2, Lq, NL), jnp.float32),
            pltpu.VMEM((2, Lq, NL), jnp.float32),
            pltpu.VMEM((8, CW), jnp.float32),
            pltpu.VMEM((8, CW), jnp.int32),
            pltpu.SemaphoreType.DMA,
            pltpu.SemaphoreType.DMA,
            pltpu.SemaphoreType.DMA,
            pltpu.SemaphoreType.DMA,
        ],
    )
    def agg(v_hbm, w_hbm, d_hbm, out_hbm, vbuf, obuf, wbuf, dbuf,
            si0, si1, so0, so1):
        wid = lax.axis_index("s") * NC + lax.axis_index("c")
        bn = wid // (C // CW)
        chb = (wid % (C // CW)) * CW
        lanes = jax.lax.broadcasted_iota(jnp.int32, (NL,), 0)
        sin = [si0, si1]
        sout = [so0, so1]
        pltpu.sync_copy(w_hbm.at[bn, :, pl.ds(chb, CW)], wbuf)
        pltpu.sync_copy(d_hbm.at[bn, :, pl.ds(chb, CW)], dbuf)

        def vin(t, buf):
            return pltpu.make_async_copy(
                v_hbm.at[bn, :, pl.ds(chb + t * NL, NL)],
                vbuf.at[buf], sin[buf])

        def vout(t, buf):
            return pltpu.make_async_copy(
                obuf.at[buf],
                out_hbm.at[bn, :, pl.ds(chb + t * NL, NL)], sout[buf])

        vin(0, 0).start()
        U = 8
        for t in range(tasks_per_w):
            cur = t % 2
            if t + 1 < tasks_per_w:
                vin(t + 1, 1 - cur).start()
            vin(t, cur).wait()
            if t >= 2:
                vout(t - 2, cur).wait()
            wv = [wbuf[i, pl.ds(t * NL, NL)] for i in range(TOPK)]
            vb = vbuf.at[cur]
            ob = obuf.at[cur]

            def group(g, idxs):
                base = g * U
                for u in range(U):
                    acc = None
                    for i in range(TOPK):
                        ix = jnp.bitwise_and(idxs[i], L - 1)
                        gt = plsc.load_gather(vb, [ix, lanes])
                        acc = gt * wv[i] if acc is None else acc + gt * wv[i]
                    ob[base + u] = acc
                    idxs = tuple(x + 1 for x in idxs)
                return idxs

            lax.fori_loop(0, Lq // U, group,
                          tuple(dbuf[i, pl.ds(t * NL, NL)]
                                for i in range(TOPK)))
            vout(t, cur).start()
        vout(tasks_per_w - 2, 0 if tasks_per_w % 2 == 0 else 1).wait()
        vout(tasks_per_w - 1, 1 if tasks_per_w % 2 == 0 else 0).wait()

    return agg(v, w, d)


def kernel(queries, keys, values, attn_mask):
    B, N, Lq, H, E = queries.shape
    C = H * E
    BN = B * N
    q = queries.reshape(BN, Lq, C)
    k = keys.reshape(BN, Lq, C)
    v = values.reshape(BN, Lq, C)

    corr, w, d = _corr_topk(q, k)
    vout = _delay_agg(v, w, d)

    V = vout.reshape(B, N, Lq, H, E)
    corr_t = corr.reshape(B, N, Lq, H, E)
    return (V, corr_t)


# speedup vs baseline: 1.3462x; 1.0062x over previous
import functools
import numpy as np
import jax
import jax.numpy as jnp
from jax import lax
from jax.experimental import pallas as pl
from jax.experimental.pallas import tpu as pltpu
from jax.experimental.pallas import tpu_sc as plsc

L = 1024
F = 520
TOPK = 6
NEG = -3.0e38


H2 = L // 2
FE = 264
FO = 256


def _dft_mats():
    j = np.arange(H2, dtype=np.int64)
    g = np.arange(FE, dtype=np.int64)
    h = np.arange(FO, dtype=np.int64)
    tau = np.arange(H2, dtype=np.int64)
    ev = np.minimum(g, 256)
    me = (np.outer(ev, j) % H2).astype(np.float64) * (2.0 * np.pi / H2)
    mo = (np.outer(2 * h + 1, j) % L).astype(np.float64) * (2.0 * np.pi / L)
    vg = (g <= 256).astype(np.float64)[:, None]
    we = (np.cos(me) * vg).astype(np.float32)
    ve = (-np.sin(me) * vg).astype(np.float32)
    wo = np.cos(mo).astype(np.float32)
    vo = (-np.sin(mo)).astype(np.float32)
    wf_e = np.where((ev == 0) | (ev == 256), 1.0, 2.0) / L
    te = (np.outer(tau, ev) % H2).astype(np.float64) * (2.0 * np.pi / H2)
    to = (np.outer(tau, 2 * h + 1) % L).astype(np.float64) * (2.0 * np.pi / L)
    ice = (np.cos(te) * wf_e[None, :] * vg.T).astype(np.float32)
    ise = (-np.sin(te) * wf_e[None, :] * vg.T).astype(np.float32)
    ico = (np.cos(to) * (2.0 / L)).astype(np.float32)
    iso = (-np.sin(to) * (2.0 / L)).astype(np.float32)
    return we, ve, wo, vo, ice, ise, ico, iso


_MATS = _dft_mats()


def _split(x):
    hi = x.astype(jnp.bfloat16)
    lo = (x - hi.astype(jnp.float32)).astype(jnp.bfloat16)
    return hi, lo


def _corr_body(q_ref, k_ref, we_ref, ve_ref, wo_ref, vo_ref,
               ice_ref, ise_ref, ico_ref, iso_ref,
               corr_ref, w_ref, d_ref):
    C = q_ref.shape[-1]
    dot = functools.partial(
        jax.lax.dot_general,
        dimension_numbers=(((1,), (0,)), ((), ())),
        preferred_element_type=jnp.float32)

    def dot3(a, b):
        ah, al = _split(a)
        bh, bl = _split(b)
        return dot(ah, bh) + (dot(ah, bl) + dot(al, bh))

    SB = C // 2
    for sb in range(2):
        sl = pl.ds(sb * SB, SB)
        q = q_ref[0, :, sl]
        k = k_ref[0, :, sl]

        qp = q[:H2] + q[H2:]
        qm = q[:H2] - q[H2:]
        kp = k[:H2] + k[H2:]
        km = k[:H2] - k[H2:]
        qre = dot3(we_ref[...], qp)
        qie = dot3(ve_ref[...], qp)
        qro = dot3(wo_ref[...], qm)
        qio = dot3(vo_ref[...], qm)
        kre = dot3(we_ref[...], kp)
        kie = dot3(ve_ref[...], kp)
        kro = dot3(wo_ref[...], km)
        kio = dot3(vo_ref[...], km)
        pre = qre * kre + qie * kie
        pie = qie * kre - qre * kie
        pro = qro * kro + qio * kio
        pio = qio * kro - qro * kio
        ue = dot3(ice_ref[...], pre) + dot3(ise_ref[...], pie)
        uo = dot3(ico_ref[...], pro) + dot3(iso_ref[...], pio)
        corr = jnp.concatenate([ue + uo, ue - uo], axis=0)
        corr_ref[0, :, sl] = corr

        riota = jax.lax.broadcasted_iota(jnp.int32, (L, SB), 0)
        c = corr
        tops, delays = [], []
        for _ in range(TOPK):
            m = jnp.max(c, axis=0, keepdims=True)
            idx = jnp.min(jnp.where(c == m, riota, L), axis=0,
                          keepdims=True)
            c = jnp.where(riota == idx, NEG, c)
            tops.append(m)
            delays.append(idx)

        es = [jnp.exp(w - tops[0]) for w in tops]
        tot = es[0]
        for e in es[1:]:
            tot = tot + e
        inv = 1.0 / tot

        zero_f = jnp.zeros((2, SB), jnp.float32)
        zero_i = jnp.zeros((2, SB), jnp.int32)
        w_ref[0, :, sl] = jnp.concatenate(
            [e * inv for e in es] + [zero_f], axis=0)
        d_ref[0, :, sl] = jnp.concatenate(delays + [zero_i], axis=0)


def _corr_topk(q, k):
    BN, Lq, C = q.shape
    CB = 512
    blk = lambda i, j: (i, 0, j)
    fix = lambda i, j: (0, 0)
    return pl.pallas_call(
        _corr_body,
        grid=(BN, C // CB),
        in_specs=[
            pl.BlockSpec((1, L, CB), blk),
            pl.BlockSpec((1, L, CB), blk),
        ] + [pl.BlockSpec(m.shape, fix) for m in _MATS],
        out_specs=[
            pl.BlockSpec((1, L, CB), blk),
            pl.BlockSpec((1, 8, CB), blk),
            pl.BlockSpec((1, 8, CB), blk),
        ],
        out_shape=[
            jax.ShapeDtypeStruct((BN, L, C), jnp.float32),
            jax.ShapeDtypeStruct((BN, 8, C), jnp.float32),
            jax.ShapeDtypeStruct((BN, 8, C), jnp.int32),
        ],
    )(q, k, *[jnp.asarray(m) for m in _MATS])


def _delay_agg(v, w, d):
    BN, Lq, C = v.shape
    info = plsc.get_sparse_core_info()
    NC, NS, NL = info.num_cores, info.num_subcores, info.num_lanes
    NW = NC * NS
    n_chunks = C // NL
    n_tasks = BN * n_chunks
    tasks_per_w = n_tasks // NW
    mesh = plsc.VectorSubcoreMesh(core_axis_name="c", subcore_axis_name="s")

    CW = tasks_per_w * NL

    @functools.partial(
        pl.kernel,
        mesh=mesh,
        compiler_params=pltpu.CompilerParams(
            use_tc_tiling_on_sc=False, needs_layout_passes=False),
        out_type=jax.ShapeDtypeStruct((BN, Lq, C), jnp.float32),
        scratch_types=[
            pltpu.VMEM((2, Lq, NL), jnp.float32),
            pltpu.VMEM((2, Lq, NL), jnp.float32),
            pltpu.VMEM((8, CW), jnp.float32),
            pltpu.VMEM((8, CW), jnp.int32),
            pltpu.SemaphoreType.DMA,
            pltpu.SemaphoreType.DMA,
            pltpu.SemaphoreType.DMA,
            pltpu.SemaphoreType.DMA,
        ],
    )
    def agg(v_hbm, w_hbm, d_hbm, out_hbm, vbuf, obuf, wbuf, dbuf,
            si0, si1, so0, so1):
        wid = lax.axis_index("s") * NC + lax.axis_index("c")
        bn = wid // (C // CW)
        chb = (wid % (C // CW)) * CW
        lanes = jax.lax.broadcasted_iota(jnp.int32, (NL,), 0)
        sin = [si0, si1]
        sout = [so0, so1]
        pltpu.sync_copy(w_hbm.at[bn, :, pl.ds(chb, CW)], wbuf)
        pltpu.sync_copy(d_hbm.at[bn, :, pl.ds(chb, CW)], dbuf)

        def vin(t, buf):
            return pltpu.make_async_copy(
                v_hbm.at[bn, :, pl.ds(chb + t * NL, NL)],
                vbuf.at[buf], sin[buf])

        def vout(t, buf):
            return pltpu.make_async_copy(
                obuf.at[buf],
                out_hbm.at[bn, :, pl.ds(chb + t * NL, NL)], sout[buf])

        vin(0, 0).start()
        U = 16
        for t in range(tasks_per_w):
            cur = t % 2
            if t + 1 < tasks_per_w:
                vin(t + 1, 1 - cur).start()
            vin(t, cur).wait()
            if t >= 2:
                vout(t - 2, cur).wait()
            wv = [wbuf[i, pl.ds(t * NL, NL)] for i in range(TOPK)]
            vb = vbuf.at[cur]
            ob = obuf.at[cur]

            def group(g, idxs):
                base = g * U
                for u in range(U):
                    acc = None
                    for i in range(TOPK):
                        ix = jnp.bitwise_and(idxs[i], L - 1)
                        gt = plsc.load_gather(vb, [ix, lanes])
                        acc = gt * wv[i] if acc is None else acc + gt * wv[i]
                    ob[base + u] = acc
                    idxs = tuple(x + 1 for x in idxs)
                return idxs

            lax.fori_loop(0, Lq // U, group,
                          tuple(dbuf[i, pl.ds(t * NL, NL)]
                                for i in range(TOPK)))
            vout(t, cur).start()
        vout(tasks_per_w - 2, 0 if tasks_per_w % 2 == 0 else 1).wait()
        vout(tasks_per_w - 1, 1 if tasks_per_w % 2 == 0 else 0).wait()

    return agg(v, w, d)


def kernel(queries, keys, values, attn_mask):
    B, N, Lq, H, E = queries.shape
    C = H * E
    BN = B * N
    q = queries.reshape(BN, Lq, C)
    k = keys.reshape(BN, Lq, C)
    v = values.reshape(BN, Lq, C)

    corr, w, d = _corr_topk(q, k)
    vout = _delay_agg(v, w, d)

    V = vout.reshape(B, N, Lq, H, E)
    corr_t = corr.reshape(B, N, Lq, H, E)
    return (V, corr_t)
